# baseline (speedup 1.0000x reference)
import functools

import jax
import jax.numpy as jnp
from jax.experimental import pallas as pl
import jax.experimental.pallas.tpu as pltpu

_NCH = 85
_NSEM = 8


def _copy_body(x_hbm, o_hbm, sems, *, n_steps):
    def copy(s):
        slot = jax.lax.rem(s, _NSEM)
        return pltpu.make_async_copy(
            x_hbm.at[s], o_hbm.at[s], sems.at[slot]
        )

    for i in range(_NSEM):
        copy(jnp.int32(i)).start()

    def step(s, carry):
        copy(s).wait()

        @pl.when(s + _NSEM < n_steps)
        def _():
            copy(s + _NSEM).start()

        return carry

    jax.lax.fori_loop(0, n_steps, step, 0)


def kernel(raw, img_size):
    n_b = raw.shape[0]
    n_g = raw.shape[2]
    n_a = raw.shape[1] // _NCH
    n_hw = n_g * n_g

    n_chunks = 96
    rr = raw.reshape(n_chunks, (n_b * n_a * _NCH * n_hw) // n_chunks)

    out = pl.pallas_call(
        functools.partial(_copy_body, n_steps=n_chunks),
        in_specs=[pl.BlockSpec(memory_space=pl.ANY)],
        out_specs=pl.BlockSpec(memory_space=pl.ANY),
        out_shape=jax.ShapeDtypeStruct(rr.shape, jnp.float32),
        scratch_shapes=[
            pltpu.SemaphoreType.DMA((_NSEM,)),
        ],
    )(rr)
    return out.reshape(n_b, n_a * n_hw, _NCH)


# speedup vs baseline: 2.4077x; 2.4077x over previous
import jax
import jax.numpy as jnp
from jax import lax
from jax.experimental import pallas as pl
from jax.experimental.pallas import tpu as pltpu
from jax.experimental.pallas import tpu_sc as plsc

_NCH = 85
_N_CLS = 80
_CHUNK = 512
_NW = 32


def _full(v, dtype=jnp.int32):
    return jnp.full((16,), v, dtype)


def _sc_body(x_hbm, stride_hbm, o_hbm, in_buf, out_buf, s_buf, *, n_slab, n_g):
    wid = lax.axis_index("s") * 2 + lax.axis_index("c")
    h0 = wid * _CHUNK
    pltpu.sync_copy(stride_hbm, s_buf)
    stride_v = s_buf[...]
    iota = lax.broadcasted_iota(jnp.int32, (16,), 0)

    def slab_body(slab, carry):
        pltpu.sync_copy(x_hbm.at[slab, :, pl.ds(h0, _CHUNK)], in_buf)
        a = lax.rem(slab, 3)
        aw_v = jnp.where(a == 0, _full(10.0, jnp.float32),
                         jnp.where(a == 1, _full(16.0, jnp.float32),
                                   _full(33.0, jnp.float32)))
        k1 = aw_v / stride_v

        def box_body(g, c2):
            cells = g * 16 + iota

            def gat(c):
                return plsc.load_gather(in_buf, [_full(c), cells])

            el = jnp.exp(gat(0)) * k1
            et = jnp.exp(gat(1)) * k1
            er = jnp.exp(gat(2)) * k1
            eb = jnp.exp(gat(3)) * k1
            hw = h0 + cells
            gx = (hw % n_g).astype(jnp.float32) + 0.5
            gy = (hw // n_g).astype(jnp.float32) + 0.5
            xc = (gx + (er - el) * 0.5) * stride_v
            yc = (gy + (eb - et) * 0.5) * stride_v
            wv = (el + er) * stride_v
            hv = (et + eb) * stride_v
            plsc.store_scatter(out_buf, [cells, _full(0)], xc)
            plsc.store_scatter(out_buf, [cells, _full(1)], yc)
            plsc.store_scatter(out_buf, [cells, _full(2)], wv)
            plsc.store_scatter(out_buf, [cells, _full(3)], hv)
            return c2

        lax.fori_loop(0, _CHUNK // 16, box_body, 0)

        def cls_body(g, c2):
            c = 4 + g // (_CHUNK // 16)
            i0 = (g % (_CHUNK // 16)) * 16
            cells = i0 + iota
            v = plsc.load_gather(in_buf, [_full(0) + c, cells])
            sig = 1.0 / (1.0 + jnp.exp(-v))
            plsc.store_scatter(out_buf, [cells, _full(0) + c], sig)
            return c2

        lax.fori_loop(0, (_NCH - 4) * (_CHUNK // 16), cls_body, 0)

        pltpu.sync_copy(out_buf, o_hbm.at[slab, pl.ds(h0, _CHUNK), :])
        return carry

    lax.fori_loop(0, n_slab, slab_body, 0)


def kernel(raw, img_size):
    n_b = raw.shape[0]
    n_g = raw.shape[2]
    n_a = raw.shape[1] // _NCH
    n_hw = n_g * n_g
    n_slab = n_b * n_a
    stride_f = (img_size // n_g).astype(jnp.float32) if hasattr(img_size, "astype") \
        else jnp.float32(img_size // n_g)
    stride_arr = jnp.full((16,), 1.0, jnp.float32) * stride_f

    rr = raw.reshape(n_slab, _NCH, n_hw)

    import functools
    body = pl.kernel(
        functools.partial(_sc_body, n_slab=n_slab, n_g=n_g),
        out_type=jax.ShapeDtypeStruct((n_slab, n_hw, _NCH), jnp.float32),
        mesh=plsc.VectorSubcoreMesh(core_axis_name="c", subcore_axis_name="s"),
        compiler_params=pltpu.CompilerParams(needs_layout_passes=False),
        scratch_types=[
            pltpu.VMEM((_NCH, _CHUNK), jnp.float32),
            pltpu.VMEM((_CHUNK, _NCH), jnp.float32),
            pltpu.VMEM((16,), jnp.float32),
        ],
    )
    out = body(rr, stride_arr)
    return out.reshape(n_b, n_a * n_hw, _NCH)


# speedup vs baseline: 11.8373x; 4.9164x over previous
import functools

import jax
import jax.numpy as jnp
from jax.experimental import pallas as pl
import jax.experimental.pallas.tpu as pltpu

_ANCHOR_W = (10.0, 16.0, 33.0)
_N_CLS = 80
_NCH = 5 + _N_CLS

_TILE = 8192
_NBUF = 6


def _decode_body(stride_ref, x_hbm, o_hbm, in_buf, out_buf, in_sems, out_sems,
                 *, n_steps, n_tiles, n_a, n_g):
    stride = stride_ref[0]
    tile = _TILE

    def in_copy(s):
        slot = jax.lax.rem(s, _NBUF)
        idx = s // n_tiles
        ts = jax.lax.rem(s, n_tiles)
        return pltpu.make_async_copy(
            x_hbm.at[idx, :, pl.ds(ts * tile, tile)],
            in_buf.at[slot],
            in_sems.at[slot],
        )

    def out_copy(s):
        slot = jax.lax.rem(s, _NBUF)
        idx = s // n_tiles
        ts = jax.lax.rem(s, n_tiles)
        return pltpu.make_async_copy(
            out_buf.at[slot],
            o_hbm.at[idx, pl.ds(ts * tile, tile)],
            out_sems.at[slot],
        )

    for i in range(_NBUF):
        in_copy(jnp.int32(i)).start()

    def step(s, carry):
        slot = jax.lax.rem(s, _NBUF)
        idx = s // n_tiles
        ts = jax.lax.rem(s, n_tiles)
        a = jax.lax.rem(idx, n_a)
        aw = jnp.where(a == 0, _ANCHOR_W[0],
                       jnp.where(a == 1, _ANCHOR_W[1], _ANCHOR_W[2]))

        in_copy(s).wait()

        @pl.when(s >= _NBUF)
        def _():
            out_copy(s - _NBUF).wait()

        x = in_buf[slot]
        ltrb = jnp.exp(x[0:4, :]) * (aw / stride)
        l = ltrb[0:1, :]
        tt = ltrb[1:2, :]
        r = ltrb[2:3, :]
        b = ltrb[3:4, :]
        hw = ts * tile + jax.lax.broadcasted_iota(jnp.int32, (1, tile), 1)
        gx = (hw % n_g).astype(jnp.float32) + 0.5
        gy = (hw // n_g).astype(jnp.float32) + 0.5
        xc = (gx + (r - l) * 0.5) * stride
        yc = (gy + (b - tt) * 0.5) * stride
        w = (l + r) * stride
        h = (tt + b) * stride
        sig = jax.nn.sigmoid(x[4:_NCH, :])
        out = jnp.concatenate([xc, yc, w, h, sig], axis=0)
        out_buf[slot] = out.T

        out_copy(s).start()

        @pl.when(s + _NBUF < n_steps)
        def _():
            in_copy(s + _NBUF).start()

        return carry

    jax.lax.fori_loop(0, n_steps, step, 0)

    for i in range(_NBUF):
        s = jnp.int32(n_steps - _NBUF + i)
        out_copy(s).wait()


def kernel(raw, img_size):
    n_b = raw.shape[0]
    n_g = raw.shape[2]
    n_a = raw.shape[1] // _NCH
    n_hw = n_g * n_g
    stride = jnp.asarray(img_size // n_g, jnp.float32).reshape(1)

    n_tiles = n_hw // _TILE
    n_steps = n_b * n_a * n_tiles
    rr = raw.reshape(n_b * n_a, _NCH, n_hw)

    out = pl.pallas_call(
        functools.partial(_decode_body, n_steps=n_steps, n_tiles=n_tiles,
                          n_a=n_a, n_g=n_g),
        in_specs=[
            pl.BlockSpec(memory_space=pltpu.SMEM),
            pl.BlockSpec(memory_space=pl.ANY),
        ],
        out_specs=pl.BlockSpec(memory_space=pl.ANY),
        out_shape=jax.ShapeDtypeStruct((n_b * n_a, n_hw, _NCH), jnp.float32),
        scratch_shapes=[
            pltpu.VMEM((_NBUF, _NCH, _TILE), jnp.float32),
            pltpu.VMEM((_NBUF, _TILE, _NCH), jnp.float32),
            pltpu.SemaphoreType.DMA((_NBUF,)),
            pltpu.SemaphoreType.DMA((_NBUF,)),
        ],
    )(stride, rr)
    return out.reshape(n_b, n_a * n_hw, _NCH)


# speedup vs baseline: 11.8618x; 1.0021x over previous
import functools

import jax
import jax.numpy as jnp
from jax.experimental import pallas as pl
import jax.experimental.pallas.tpu as pltpu

_ANCHOR_W = (10.0, 16.0, 33.0)
_N_CLS = 80
_NCH = 5 + _N_CLS

_TILE = 16384
_NBUF = 3


def _decode_body(stride_ref, x_hbm, o_hbm, in_buf, out_buf, in_sems, out_sems,
                 *, n_steps, n_tiles, n_a, n_g):
    stride = stride_ref[0]
    tile = _TILE

    def in_copy(s):
        slot = jax.lax.rem(s, _NBUF)
        idx = s // n_tiles
        ts = jax.lax.rem(s, n_tiles)
        return pltpu.make_async_copy(
            x_hbm.at[idx, :, pl.ds(ts * tile, tile)],
            in_buf.at[slot],
            in_sems.at[slot],
        )

    def out_copy(s):
        slot = jax.lax.rem(s, _NBUF)
        idx = s // n_tiles
        ts = jax.lax.rem(s, n_tiles)
        return pltpu.make_async_copy(
            out_buf.at[slot],
            o_hbm.at[idx, pl.ds(ts * tile, tile)],
            out_sems.at[slot],
        )

    for i in range(_NBUF):
        in_copy(jnp.int32(i)).start()

    def step(s, carry):
        slot = jax.lax.rem(s, _NBUF)
        idx = s // n_tiles
        ts = jax.lax.rem(s, n_tiles)
        a = jax.lax.rem(idx, n_a)
        aw = jnp.where(a == 0, _ANCHOR_W[0],
                       jnp.where(a == 1, _ANCHOR_W[1], _ANCHOR_W[2]))

        in_copy(s).wait()

        @pl.when(s >= _NBUF)
        def _():
            out_copy(s - _NBUF).wait()

        x = in_buf[slot]
        ltrb = jnp.exp(x[0:4, :]) * (aw / stride)
        l = ltrb[0:1, :]
        tt = ltrb[1:2, :]
        r = ltrb[2:3, :]
        b = ltrb[3:4, :]
        hw = ts * tile + jax.lax.broadcasted_iota(jnp.int32, (1, tile), 1)
        gx = (hw % n_g).astype(jnp.float32) + 0.5
        gy = (hw // n_g).astype(jnp.float32) + 0.5
        xc = (gx + (r - l) * 0.5) * stride
        yc = (gy + (b - tt) * 0.5) * stride
        w = (l + r) * stride
        h = (tt + b) * stride
        sig = jax.nn.sigmoid(x[4:_NCH, :])
        out = jnp.concatenate([xc, yc, w, h, sig], axis=0)
        out_buf[slot] = out.T

        out_copy(s).start()

        @pl.when(s + _NBUF < n_steps)
        def _():
            in_copy(s + _NBUF).start()

        return carry

    jax.lax.fori_loop(0, n_steps, step, 0)

    for i in range(_NBUF):
        s = jnp.int32(n_steps - _NBUF + i)
        out_copy(s).wait()


def kernel(raw, img_size):
    n_b = raw.shape[0]
    n_g = raw.shape[2]
    n_a = raw.shape[1] // _NCH
    n_hw = n_g * n_g
    stride = jnp.asarray(img_size // n_g, jnp.float32).reshape(1)

    n_tiles = n_hw // _TILE
    n_steps = n_b * n_a * n_tiles
    rr = raw.reshape(n_b * n_a, _NCH, n_hw)

    out = pl.pallas_call(
        functools.partial(_decode_body, n_steps=n_steps, n_tiles=n_tiles,
                          n_a=n_a, n_g=n_g),
        in_specs=[
            pl.BlockSpec(memory_space=pltpu.SMEM),
            pl.BlockSpec(memory_space=pl.ANY),
        ],
        out_specs=pl.BlockSpec(memory_space=pl.ANY),
        out_shape=jax.ShapeDtypeStruct((n_b * n_a, n_hw, _NCH), jnp.float32),
        scratch_shapes=[
            pltpu.VMEM((_NBUF, _NCH, _TILE), jnp.float32),
            pltpu.VMEM((_NBUF, _TILE, _NCH), jnp.float32),
            pltpu.SemaphoreType.DMA((_NBUF,)),
            pltpu.SemaphoreType.DMA((_NBUF,)),
        ],
    )(stride, rr)
    return out.reshape(n_b, n_a * n_hw, _NCH)
